# baseline (speedup 1.0000x reference)
import functools

import jax
import jax.numpy as jnp
from jax import lax
from jax.experimental import pallas as pl
from jax.experimental.pallas import tpu as pltpu
from jax.experimental.pallas import tpu_sc as plsc

NC = 2
NS = 16
SCATTER_B = 80
CHUNK = 2000


def _sc_scatter_partials(edge_attr, edge_index, zeros_init, ones_rows, n_pad):
    e, de = edge_attr.shape
    nw = NC * NS
    epw = e // nw
    n_chunks = epw // CHUNK
    n_batches = CHUNK // SCATTER_B
    batches_pw = epw // SCATTER_B
    rows_per_tile = n_pad // NS

    mesh = plsc.VectorSubcoreMesh(
        core_axis_name="c", subcore_axis_name="s",
        num_cores=NC, num_subcores=NS)

    @functools.partial(
        pl.kernel,
        out_type=[
            jax.ShapeDtypeStruct((NC, n_pad, de), jnp.float32),
            jax.ShapeDtypeStruct((NC, n_pad, de), jnp.float32),
        ],
        mesh=mesh,
        compiler_params=pltpu.CompilerParams(use_tc_tiling_on_sc=False),
        scratch_types=[
            pltpu.VMEM_SHARED((n_pad, de), jnp.float32),
            pltpu.VMEM_SHARED((n_pad, de), jnp.float32),
            pltpu.VMEM((CHUNK, de), jnp.float32),
            pltpu.VMEM((CHUNK,), jnp.int32),
            pltpu.VMEM((CHUNK, de), jnp.float32),
        ],
    )
    def body(attr_hbm, idx_hbm, zeros_hbm, ones_hbm, psum, pcnt,
             sums_sh, cnt_sh, attr_v, idx_v, ones_v):
        c = lax.axis_index("c")
        s = lax.axis_index("s")
        w = c * NS + s

        rbase = s * rows_per_tile
        pltpu.sync_copy(zeros_hbm,
                        sums_sh.at[pl.ds(rbase, rows_per_tile)])
        pltpu.sync_copy(zeros_hbm,
                        cnt_sh.at[pl.ds(rbase, rows_per_tile)])
        pltpu.sync_copy(ones_hbm, ones_v)
        plsc.subcore_barrier()

        ebase = w * epw
        for k in range(n_chunks):
            pltpu.sync_copy(attr_hbm.at[pl.ds(ebase + k * CHUNK, CHUNK)],
                            attr_v)
            pltpu.sync_copy(idx_hbm.at[1, pl.ds(ebase + k * CHUNK, CHUNK)],
                            idx_v)
            pltpu.sync_copy(attr_v, sums_sh.at[idx_v], add=True)
            pltpu.sync_copy(ones_v, cnt_sh.at[idx_v], add=True)

        plsc.subcore_barrier()

        pltpu.sync_copy(sums_sh.at[pl.ds(rbase, rows_per_tile)],
                        psum.at[c, pl.ds(rbase, rows_per_tile)])
        pltpu.sync_copy(cnt_sh.at[pl.ds(rbase, rows_per_tile)],
                        pcnt.at[c, pl.ds(rbase, rows_per_tile)])

    return body(edge_attr, edge_index, zeros_init, ones_rows)


def _tc_mlp_kernel(x_ref, psum_ref, pcnt_ref, batch_ref, u_ref,
                   w1_ref, b1_ref, w2_ref, b2_ref, out_ref, *, d, de, g):
    sums = psum_ref[0] + psum_ref[1]
    cnt = pcnt_ref[0, :, 0:1] + pcnt_ref[1, :, 0:1]
    agg = sums / jnp.maximum(cnt, 1.0)

    onehot = (batch_ref[...] ==
              lax.broadcasted_iota(jnp.int32, (1, g), 1)).astype(jnp.float32)
    uw = jnp.dot(u_ref[...], w1_ref[d + de:, :],
                 preferred_element_type=jnp.float32)
    h = jnp.dot(x_ref[...], w1_ref[:d, :],
                preferred_element_type=jnp.float32)
    h += jnp.dot(agg, w1_ref[d:d + de, :],
                 preferred_element_type=jnp.float32)
    h += jnp.dot(onehot, uw, preferred_element_type=jnp.float32)
    h = jnp.maximum(h + b1_ref[...], 0.0)
    out = jnp.dot(h, w2_ref[...], preferred_element_type=jnp.float32)
    out_ref[...] = jnp.maximum(out + b2_ref[...], 0.0)


def kernel(x, edge_index, edge_attr, u, batch, W1, b1, W2, b2):
    n, d = x.shape
    e, de = edge_attr.shape
    g = u.shape[0]
    lat = W2.shape[1]

    n_pad = -(-n // (8 * NS)) * (8 * NS)
    zeros_init = jnp.zeros((n_pad // NS, de), dtype=jnp.float32)
    ones_rows = jnp.ones((CHUNK, de), dtype=jnp.float32)

    psum, pcnt = _sc_scatter_partials(edge_attr, edge_index, zeros_init,
                                      ones_rows, n_pad)

    bn = 1000
    grid = n // bn
    tc = pl.pallas_call(
        functools.partial(_tc_mlp_kernel, d=d, de=de, g=g),
        grid=(grid,),
        in_specs=[
            pl.BlockSpec((bn, d), lambda i: (i, 0)),
            pl.BlockSpec((NC, bn, de), lambda i: (0, i, 0)),
            pl.BlockSpec((NC, bn, de), lambda i: (0, i, 0)),
            pl.BlockSpec((bn, 1), lambda i: (i, 0)),
            pl.BlockSpec((g, d), lambda i: (0, 0)),
            pl.BlockSpec(W1.shape, lambda i: (0, 0)),
            pl.BlockSpec((1, lat), lambda i: (0, 0)),
            pl.BlockSpec(W2.shape, lambda i: (0, 0)),
            pl.BlockSpec((1, lat), lambda i: (0, 0)),
        ],
        out_specs=pl.BlockSpec((bn, lat), lambda i: (i, 0)),
        out_shape=jax.ShapeDtypeStruct((n, lat), jnp.float32),
    )
    return tc(x, psum, pcnt, batch.reshape(n, 1), u,
              W1, b1.reshape(1, lat), W2, b2.reshape(1, lat))


# speedup vs baseline: 1.0214x; 1.0214x over previous
import functools

import jax
import jax.numpy as jnp
from jax import lax
from jax.experimental import pallas as pl
from jax.experimental.pallas import tpu as pltpu
from jax.experimental.pallas import tpu_sc as plsc

NC = 2
NS = 16
SCATTER_B = 80
CHUNK = 2000


def _tc_transpose_kernel(in_ref, out_ref, scr, *, bk, de):
    scr[...] = in_ref[...].T
    n8 = 128 // de
    rows = bk // n8
    for j in range(n8):
        out_ref[:, j * de:(j + 1) * de] = scr[pl.Slice(j, rows, n8), :]


def _edge_rows_from_feature_major(ea_t, e, de):
    bk = 3200
    grid = e // bk
    tp = pl.pallas_call(
        functools.partial(_tc_transpose_kernel, bk=bk, de=de),
        grid=(grid,),
        in_specs=[pl.BlockSpec((de, bk), lambda i: (0, i))],
        out_specs=pl.BlockSpec((bk * de // 128, 128), lambda i: (i, 0)),
        out_shape=jax.ShapeDtypeStruct((e * de // 128, 128), jnp.float32),
        scratch_shapes=[pltpu.VMEM((bk, de), jnp.float32)],
    )
    return tp(ea_t).reshape(e, de)


def _sc_scatter_partials(ea_rows, dest, zeros_init, ones_rows, n_pad, de):
    e = dest.shape[0]
    nw = NC * NS
    epw = e // nw
    n_chunks = epw // CHUNK
    n_batches = CHUNK // SCATTER_B
    batches_pw = epw // SCATTER_B
    rows_per_tile = n_pad // NS

    mesh = plsc.VectorSubcoreMesh(
        core_axis_name="c", subcore_axis_name="s",
        num_cores=NC, num_subcores=NS)

    @functools.partial(
        pl.kernel,
        out_type=[
            jax.ShapeDtypeStruct((NC, n_pad, de), jnp.float32),
            jax.ShapeDtypeStruct((NC, n_pad, de), jnp.float32),
        ],
        mesh=mesh,
        compiler_params=pltpu.CompilerParams(use_tc_tiling_on_sc=False),
        scratch_types=[
            pltpu.VMEM_SHARED((n_pad, de), jnp.float32),
            pltpu.VMEM_SHARED((n_pad, de), jnp.float32),
            pltpu.VMEM((CHUNK, de), jnp.float32),
            pltpu.VMEM((CHUNK,), jnp.int32),
            pltpu.VMEM((CHUNK, de), jnp.float32),
        ],
    )
    def body(attr_hbm, idx_hbm, zeros_hbm, ones_hbm, psum, pcnt,
             sums_sh, cnt_sh, attr_v, idx_v, ones_v):
        c = lax.axis_index("c")
        s = lax.axis_index("s")
        w = c * NS + s

        rbase = s * rows_per_tile
        pltpu.sync_copy(zeros_hbm,
                        sums_sh.at[pl.ds(rbase, rows_per_tile)])
        pltpu.sync_copy(zeros_hbm,
                        cnt_sh.at[pl.ds(rbase, rows_per_tile)])
        pltpu.sync_copy(ones_hbm, ones_v)
        plsc.subcore_barrier()

        ebase = w * epw
        for k in range(n_chunks):
            pltpu.sync_copy(attr_hbm.at[pl.ds(ebase + k * CHUNK, CHUNK)],
                            attr_v)
            pltpu.sync_copy(idx_hbm.at[pl.ds(ebase + k * CHUNK, CHUNK)],
                            idx_v)
            pltpu.sync_copy(attr_v, sums_sh.at[idx_v], add=True)
            pltpu.sync_copy(ones_v, cnt_sh.at[idx_v], add=True)

        plsc.subcore_barrier()

        pltpu.sync_copy(sums_sh.at[pl.ds(rbase, rows_per_tile)],
                        psum.at[c, pl.ds(rbase, rows_per_tile)])
        pltpu.sync_copy(cnt_sh.at[pl.ds(rbase, rows_per_tile)],
                        pcnt.at[c, pl.ds(rbase, rows_per_tile)])

    return body(ea_rows, dest, zeros_init, ones_rows)


def _tc_mlp_kernel(x_ref, psum_ref, pcnt_ref, batch_ref, u_ref,
                   w1_ref, b1_ref, w2_ref, b2_ref, out_ref, *, d, de, g):
    sums = psum_ref[0] + psum_ref[1]
    cnt = pcnt_ref[0, :, 0:1] + pcnt_ref[1, :, 0:1]
    agg = sums / jnp.maximum(cnt, 1.0)

    onehot = (batch_ref[...] ==
              lax.broadcasted_iota(jnp.int32, (1, g), 1)).astype(jnp.float32)
    uw = jnp.dot(u_ref[...], w1_ref[d + de:, :],
                 preferred_element_type=jnp.float32)
    h = jnp.dot(x_ref[...], w1_ref[:d, :],
                preferred_element_type=jnp.float32)
    h += jnp.dot(agg, w1_ref[d:d + de, :],
                 preferred_element_type=jnp.float32)
    h += jnp.dot(onehot, uw, preferred_element_type=jnp.float32)
    h = jnp.maximum(h + b1_ref[...], 0.0)
    out = jnp.dot(h, w2_ref[...], preferred_element_type=jnp.float32)
    out_ref[...] = jnp.maximum(out + b2_ref[...], 0.0)


def kernel(x, edge_index, edge_attr, u, batch, W1, b1, W2, b2):
    n, d = x.shape
    e, de = edge_attr.shape
    g = u.shape[0]
    lat = W2.shape[1]

    n_pad = -(-n // (8 * NS)) * (8 * NS)
    zeros_init = jnp.zeros((n_pad // NS, de), dtype=jnp.float32)
    ones_rows = jnp.ones((CHUNK, de), dtype=jnp.float32)

    ea_rows = _edge_rows_from_feature_major(jnp.swapaxes(edge_attr, 0, 1),
                                            e, de)
    psum, pcnt = _sc_scatter_partials(ea_rows, edge_index[1], zeros_init,
                                      ones_rows, n_pad, de)

    bn = 1000
    grid = n // bn
    tc = pl.pallas_call(
        functools.partial(_tc_mlp_kernel, d=d, de=de, g=g),
        grid=(grid,),
        in_specs=[
            pl.BlockSpec((bn, d), lambda i: (i, 0)),
            pl.BlockSpec((NC, bn, de), lambda i: (0, i, 0)),
            pl.BlockSpec((NC, bn, de), lambda i: (0, i, 0)),
            pl.BlockSpec((bn, 1), lambda i: (i, 0)),
            pl.BlockSpec((g, d), lambda i: (0, 0)),
            pl.BlockSpec(W1.shape, lambda i: (0, 0)),
            pl.BlockSpec((1, lat), lambda i: (0, 0)),
            pl.BlockSpec(W2.shape, lambda i: (0, 0)),
            pl.BlockSpec((1, lat), lambda i: (0, 0)),
        ],
        out_specs=pl.BlockSpec((bn, lat), lambda i: (i, 0)),
        out_shape=jax.ShapeDtypeStruct((n, lat), jnp.float32),
    )
    return tc(x, psum, pcnt, batch.reshape(n, 1), u,
              W1, b1.reshape(1, lat), W2, b2.reshape(1, lat))


# speedup vs baseline: 1.0874x; 1.0647x over previous
import functools

import jax
import jax.numpy as jnp
from jax import lax
from jax.experimental import pallas as pl
from jax.experimental.pallas import tpu as pltpu
from jax.experimental.pallas import tpu_sc as plsc

NC = 2
NS = 16
SCATTER_B = 80
CHUNK = 2000


def _sc_scatter_partials(ea_t, dest, zeros_init, ones_rows, n_pad, de):
    e = dest.shape[0]
    nw = NC * NS
    epw = e // nw
    n_chunks = epw // CHUNK
    n_batches = CHUNK // SCATTER_B
    batches_pw = epw // SCATTER_B
    rows_per_tile = n_pad // NS

    mesh = plsc.VectorSubcoreMesh(
        core_axis_name="c", subcore_axis_name="s",
        num_cores=NC, num_subcores=NS)

    @functools.partial(
        pl.kernel,
        out_type=[
            jax.ShapeDtypeStruct((NC, n_pad, de), jnp.float32),
            jax.ShapeDtypeStruct((NC, n_pad, de), jnp.float32),
        ],
        mesh=mesh,
        compiler_params=pltpu.CompilerParams(use_tc_tiling_on_sc=False,
                                             needs_layout_passes=False),
        scratch_types=[
            pltpu.VMEM_SHARED((n_pad, de), jnp.float32),
            pltpu.VMEM_SHARED((n_pad, de), jnp.float32),
            pltpu.VMEM((de, CHUNK), jnp.float32),
            pltpu.VMEM((CHUNK, de), jnp.float32),
            pltpu.VMEM((CHUNK,), jnp.int32),
            pltpu.VMEM((CHUNK, de), jnp.float32),
        ],
    )
    def body(attr_hbm, idx_hbm, zeros_hbm, ones_hbm, psum, pcnt,
             sums_sh, cnt_sh, col_v, attr_v, idx_v, ones_v):
        c = lax.axis_index("c")
        s = lax.axis_index("s")
        w = c * NS + s

        rbase = s * rows_per_tile
        pltpu.sync_copy(zeros_hbm,
                        sums_sh.at[pl.ds(rbase, rows_per_tile)])
        pltpu.sync_copy(zeros_hbm,
                        cnt_sh.at[pl.ds(rbase, rows_per_tile)])
        pltpu.sync_copy(ones_hbm, ones_v)
        plsc.subcore_barrier()

        ebase = w * epw
        feat = lax.iota(jnp.int32, 16)
        unroll = 8
        for k in range(n_chunks):
            pltpu.sync_copy(attr_hbm.at[:, pl.ds(ebase + k * CHUNK, CHUNK)],
                            col_v)
            pltpu.sync_copy(idx_hbm.at[pl.ds(ebase + k * CHUNK, CHUNK)],
                            idx_v)

            def assemble(i, carry):
                for uu in range(unroll):
                    col = i * unroll + uu
                    row = plsc.load_gather(
                        col_v, [feat, jnp.full((16,), col, jnp.int32)])
                    attr_v[col] = row
                return carry

            lax.fori_loop(0, CHUNK // unroll, assemble, 0)
            pltpu.sync_copy(attr_v, sums_sh.at[idx_v], add=True)
            pltpu.sync_copy(ones_v, cnt_sh.at[idx_v], add=True)

        plsc.subcore_barrier()

        pltpu.sync_copy(sums_sh.at[pl.ds(rbase, rows_per_tile)],
                        psum.at[c, pl.ds(rbase, rows_per_tile)])
        pltpu.sync_copy(cnt_sh.at[pl.ds(rbase, rows_per_tile)],
                        pcnt.at[c, pl.ds(rbase, rows_per_tile)])

    return body(ea_t, dest, zeros_init, ones_rows)


def _tc_mlp_kernel(x_ref, psum_ref, pcnt_ref, batch_ref, u_ref,
                   w1_ref, b1_ref, w2_ref, b2_ref, out_ref, *, d, de, g):
    sums = psum_ref[0] + psum_ref[1]
    cnt = pcnt_ref[0, :, 0:1] + pcnt_ref[1, :, 0:1]
    agg = sums / jnp.maximum(cnt, 1.0)

    onehot = (batch_ref[...] ==
              lax.broadcasted_iota(jnp.int32, (1, g), 1)).astype(jnp.float32)
    uw = jnp.dot(u_ref[...], w1_ref[d + de:, :],
                 preferred_element_type=jnp.float32)
    h = jnp.dot(x_ref[...], w1_ref[:d, :],
                preferred_element_type=jnp.float32)
    h += jnp.dot(agg, w1_ref[d:d + de, :],
                 preferred_element_type=jnp.float32)
    h += jnp.dot(onehot, uw, preferred_element_type=jnp.float32)
    h = jnp.maximum(h + b1_ref[...], 0.0)
    out = jnp.dot(h, w2_ref[...], preferred_element_type=jnp.float32)
    out_ref[...] = jnp.maximum(out + b2_ref[...], 0.0)


def kernel(x, edge_index, edge_attr, u, batch, W1, b1, W2, b2):
    n, d = x.shape
    e, de = edge_attr.shape
    g = u.shape[0]
    lat = W2.shape[1]

    n_pad = -(-n // (8 * NS)) * (8 * NS)
    zeros_init = jnp.zeros((n_pad // NS, de), dtype=jnp.float32)
    ones_rows = jnp.ones((CHUNK, de), dtype=jnp.float32)

    psum, pcnt = _sc_scatter_partials(jnp.swapaxes(edge_attr, 0, 1),
                                      edge_index[1], zeros_init,
                                      ones_rows, n_pad, de)

    bn = 1000
    grid = n // bn
    tc = pl.pallas_call(
        functools.partial(_tc_mlp_kernel, d=d, de=de, g=g),
        grid=(grid,),
        in_specs=[
            pl.BlockSpec((bn, d), lambda i: (i, 0)),
            pl.BlockSpec((NC, bn, de), lambda i: (0, i, 0)),
            pl.BlockSpec((NC, bn, de), lambda i: (0, i, 0)),
            pl.BlockSpec((bn, 1), lambda i: (i, 0)),
            pl.BlockSpec((g, d), lambda i: (0, 0)),
            pl.BlockSpec(W1.shape, lambda i: (0, 0)),
            pl.BlockSpec((1, lat), lambda i: (0, 0)),
            pl.BlockSpec(W2.shape, lambda i: (0, 0)),
            pl.BlockSpec((1, lat), lambda i: (0, 0)),
        ],
        out_specs=pl.BlockSpec((bn, lat), lambda i: (i, 0)),
        out_shape=jax.ShapeDtypeStruct((n, lat), jnp.float32),
    )
    return tc(x, psum, pcnt, batch.reshape(n, 1), u,
              W1, b1.reshape(1, lat), W2, b2.reshape(1, lat))


# speedup vs baseline: 1.4300x; 1.3150x over previous
import functools

import jax
import jax.numpy as jnp
from jax import lax
from jax.experimental import pallas as pl
from jax.experimental.pallas import tpu as pltpu
from jax.experimental.pallas import tpu_sc as plsc

NC = 2
NS = 16
CHUNK = 1000


def _sc_scatter_partials(ea_t, dest, zeros_init, ones_rows, n_pad, de):
    e = dest.shape[0]
    nw = NC * NS
    epw = e // nw
    n_chunks = epw // CHUNK
    rows_per_tile = n_pad // NS

    mesh = plsc.VectorSubcoreMesh(
        core_axis_name="c", subcore_axis_name="s",
        num_cores=NC, num_subcores=NS)

    @functools.partial(
        pl.kernel,
        out_type=[
            jax.ShapeDtypeStruct((NC, n_pad, de), jnp.float32),
            jax.ShapeDtypeStruct((NC, n_pad, de), jnp.float32),
        ],
        mesh=mesh,
        compiler_params=pltpu.CompilerParams(use_tc_tiling_on_sc=False,
                                             needs_layout_passes=False),
        scratch_types=[
            pltpu.VMEM_SHARED((n_pad, de), jnp.float32),
            pltpu.VMEM_SHARED((n_pad, de), jnp.float32),
            pltpu.VMEM((2, de * CHUNK), jnp.float32),
            pltpu.VMEM((2, CHUNK, de), jnp.float32),
            pltpu.VMEM((n_chunks, CHUNK), jnp.int32),
            pltpu.VMEM((CHUNK, de), jnp.float32),
            pltpu.SemaphoreType.DMA((2,)),
            pltpu.SemaphoreType.DMA((2,)),
        ],
    )
    def body(attr_hbm, idx_hbm, zeros_hbm, ones_hbm, psum, pcnt,
             sums_sh, cnt_sh, col_v, attr_v, idx_v, ones_v, lsem, ssem):
        c = lax.axis_index("c")
        s = lax.axis_index("s")
        w = c * NS + s

        rbase = s * rows_per_tile
        pltpu.sync_copy(zeros_hbm,
                        sums_sh.at[pl.ds(rbase, rows_per_tile)])
        pltpu.sync_copy(zeros_hbm,
                        cnt_sh.at[pl.ds(rbase, rows_per_tile)])
        pltpu.sync_copy(ones_hbm, ones_v)
        plsc.subcore_barrier()

        ebase = w * epw
        idx0 = lax.iota(jnp.int32, 16) * CHUNK
        unroll = 8

        def start_load(k):
            b = k % 2
            base = ebase + k * CHUNK
            hs = [pltpu.async_copy(attr_hbm.at[f, pl.ds(base, CHUNK)],
                                   col_v.at[b, pl.ds(f * CHUNK, CHUNK)],
                                   lsem.at[b])
                  for f in range(de)]
            hs.append(pltpu.async_copy(idx_hbm.at[pl.ds(base, CHUNK)],
                                       idx_v.at[k], lsem.at[b]))
            return hs

        def assemble(b):
            src = col_v.at[b]
            dst = attr_v.at[b]

            def go(i, idxvec):
                for uu in range(unroll):
                    row = plsc.load_gather(src, [idxvec])
                    dst[i * unroll + uu] = row
                    idxvec = idxvec + 1
                return idxvec

            lax.fori_loop(0, CHUNK // unroll, go, idx0)

        def start_scatter(k):
            b = k % 2
            return [pltpu.async_copy(attr_v.at[b], sums_sh.at[idx_v.at[k]],
                                     ssem.at[b], add=True),
                    pltpu.async_copy(ones_v, cnt_sh.at[idx_v.at[k]],
                                     ssem.at[b], add=True)]

        loads = {0: start_load(0)}
        scats = {}
        for k in range(n_chunks):
            b = k % 2
            for h in loads.pop(k):
                h.wait()
            if k + 1 < n_chunks:
                loads[k + 1] = start_load(k + 1)
            if k - 2 in scats:
                for h in scats.pop(k - 2):
                    h.wait()
            assemble(b)
            scats[k] = start_scatter(k)
        for hs in scats.values():
            for h in hs:
                h.wait()

        plsc.subcore_barrier()

        pltpu.sync_copy(sums_sh.at[pl.ds(rbase, rows_per_tile)],
                        psum.at[c, pl.ds(rbase, rows_per_tile)])
        pltpu.sync_copy(cnt_sh.at[pl.ds(rbase, rows_per_tile)],
                        pcnt.at[c, pl.ds(rbase, rows_per_tile)])

    return body(ea_t, dest, zeros_init, ones_rows)


def _tc_mlp_kernel(x_ref, psum_ref, pcnt_ref, batch_ref, u_ref,
                   w1_ref, b1_ref, w2_ref, b2_ref, out_ref, *, d, de, g):
    sums = psum_ref[0] + psum_ref[1]
    cnt = pcnt_ref[0, :, 0:1] + pcnt_ref[1, :, 0:1]
    agg = sums / jnp.maximum(cnt, 1.0)

    onehot = (batch_ref[...] ==
              lax.broadcasted_iota(jnp.int32, (1, g), 1)).astype(jnp.float32)
    uw = jnp.dot(u_ref[...], w1_ref[d + de:, :],
                 preferred_element_type=jnp.float32)
    h = jnp.dot(x_ref[...], w1_ref[:d, :],
                preferred_element_type=jnp.float32)
    h += jnp.dot(agg, w1_ref[d:d + de, :],
                 preferred_element_type=jnp.float32)
    h += jnp.dot(onehot, uw, preferred_element_type=jnp.float32)
    h = jnp.maximum(h + b1_ref[...], 0.0)
    out = jnp.dot(h, w2_ref[...], preferred_element_type=jnp.float32)
    out_ref[...] = jnp.maximum(out + b2_ref[...], 0.0)


def kernel(x, edge_index, edge_attr, u, batch, W1, b1, W2, b2):
    n, d = x.shape
    e, de = edge_attr.shape
    g = u.shape[0]
    lat = W2.shape[1]

    n_pad = -(-n // (8 * NS)) * (8 * NS)
    zeros_init = jnp.zeros((n_pad // NS, de), dtype=jnp.float32)
    ones_rows = jnp.ones((CHUNK, de), dtype=jnp.float32)

    psum, pcnt = _sc_scatter_partials(jnp.swapaxes(edge_attr, 0, 1),
                                      edge_index[1], zeros_init,
                                      ones_rows, n_pad, de)

    bn = 1000
    grid = n // bn
    tc = pl.pallas_call(
        functools.partial(_tc_mlp_kernel, d=d, de=de, g=g),
        grid=(grid,),
        in_specs=[
            pl.BlockSpec((bn, d), lambda i: (i, 0)),
            pl.BlockSpec((NC, bn, de), lambda i: (0, i, 0)),
            pl.BlockSpec((NC, bn, de), lambda i: (0, i, 0)),
            pl.BlockSpec((bn, 1), lambda i: (i, 0)),
            pl.BlockSpec((g, d), lambda i: (0, 0)),
            pl.BlockSpec(W1.shape, lambda i: (0, 0)),
            pl.BlockSpec((1, lat), lambda i: (0, 0)),
            pl.BlockSpec(W2.shape, lambda i: (0, 0)),
            pl.BlockSpec((1, lat), lambda i: (0, 0)),
        ],
        out_specs=pl.BlockSpec((bn, lat), lambda i: (i, 0)),
        out_shape=jax.ShapeDtypeStruct((n, lat), jnp.float32),
    )
    return tc(x, psum, pcnt, batch.reshape(n, 1), u,
              W1, b1.reshape(1, lat), W2, b2.reshape(1, lat))


# speedup vs baseline: 1.8014x; 1.2597x over previous
import functools

import jax
import jax.numpy as jnp
from jax import lax
from jax.experimental import pallas as pl
from jax.experimental.pallas import tpu as pltpu
from jax.experimental.pallas import tpu_sc as plsc

NC = 2
NS = 16
CHUNK = 1000


def _sc_scatter_partials(ea_t, dest, zeros_init, ones_rows, n_pad, de):
    e = dest.shape[0]
    nw = NC * NS
    epw = e // nw
    n_chunks = epw // CHUNK
    rows_per_tile = n_pad // NS

    mesh = plsc.VectorSubcoreMesh(
        core_axis_name="c", subcore_axis_name="s",
        num_cores=NC, num_subcores=NS)

    @functools.partial(
        pl.kernel,
        out_type=[
            jax.ShapeDtypeStruct((NC, n_pad, de), jnp.float32),
            jax.ShapeDtypeStruct((NC, n_pad, de), jnp.float32),
        ],
        mesh=mesh,
        compiler_params=pltpu.CompilerParams(use_tc_tiling_on_sc=False,
                                             needs_layout_passes=False),
        scratch_types=[
            pltpu.VMEM_SHARED((n_pad, de), jnp.float32),
            pltpu.VMEM_SHARED((n_pad, de), jnp.float32),
            pltpu.VMEM((2, de * CHUNK), jnp.float32),
            pltpu.VMEM((2, CHUNK, de), jnp.float32),
            pltpu.VMEM((n_chunks, CHUNK), jnp.int32),
            pltpu.VMEM((CHUNK, de), jnp.float32),
            pltpu.SemaphoreType.DMA((2,)),
            pltpu.SemaphoreType.DMA((2,)),
        ],
    )
    def body(attr_hbm, idx_hbm, zeros_hbm, ones_hbm, psum, pcnt,
             sums_sh, cnt_sh, col_v, attr_v, idx_v, ones_v, lsem, ssem):
        c = lax.axis_index("c")
        s = lax.axis_index("s")
        w = c * NS + s

        rbase = s * rows_per_tile
        pltpu.sync_copy(zeros_hbm,
                        sums_sh.at[pl.ds(rbase, rows_per_tile)])
        pltpu.sync_copy(zeros_hbm,
                        cnt_sh.at[pl.ds(rbase, rows_per_tile)])
        pltpu.sync_copy(ones_hbm, ones_v)
        plsc.subcore_barrier()

        ebase = w * epw
        idx0 = lax.iota(jnp.int32, 16) * CHUNK
        unroll = 8

        def start_load(k):
            b = k % 2
            base = ebase + k * CHUNK
            hs = [pltpu.async_copy(attr_hbm.at[f, pl.ds(base, CHUNK)],
                                   col_v.at[b, pl.ds(f * CHUNK, CHUNK)],
                                   lsem.at[b])
                  for f in range(de)]
            hs.append(pltpu.async_copy(idx_hbm.at[pl.ds(base, CHUNK)],
                                       idx_v.at[k], lsem.at[b]))
            return hs

        def assemble(b):
            src = col_v.at[b]
            dst = attr_v.at[b]

            def go(i, idxvec):
                rows = [plsc.load_gather(src, [idxvec + uu])
                        for uu in range(unroll)]
                for uu in range(unroll):
                    dst[i * unroll + uu] = rows[uu]
                return idxvec + unroll

            lax.fori_loop(0, CHUNK // unroll, go, idx0)

        def start_scatter(k):
            b = k % 2
            return [pltpu.async_copy(attr_v.at[b], sums_sh.at[idx_v.at[k]],
                                     ssem.at[b], add=True),
                    pltpu.async_copy(ones_v, cnt_sh.at[idx_v.at[k]],
                                     ssem.at[b], add=True)]

        loads = {0: start_load(0)}
        scats = {}
        for k in range(n_chunks):
            b = k % 2
            for h in loads.pop(k):
                h.wait()
            if k + 1 < n_chunks:
                loads[k + 1] = start_load(k + 1)
            if k - 2 in scats:
                for h in scats.pop(k - 2):
                    h.wait()
            assemble(b)
            scats[k] = start_scatter(k)
        for hs in scats.values():
            for h in hs:
                h.wait()

        plsc.subcore_barrier()

        pltpu.sync_copy(sums_sh.at[pl.ds(rbase, rows_per_tile)],
                        psum.at[c, pl.ds(rbase, rows_per_tile)])
        pltpu.sync_copy(cnt_sh.at[pl.ds(rbase, rows_per_tile)],
                        pcnt.at[c, pl.ds(rbase, rows_per_tile)])

    return body(ea_t, dest, zeros_init, ones_rows)


def _tc_pre_kernel(x_ref, batch_ref, u_ref, w1_ref, b1_ref, t1_ref,
                   *, d, de, g):
    onehot = (batch_ref[...] ==
              lax.broadcasted_iota(jnp.int32, (1, g), 1)).astype(jnp.float32)
    uw = jnp.dot(u_ref[...], w1_ref[d + de:, :],
                 preferred_element_type=jnp.float32)
    t1 = jnp.dot(x_ref[...], w1_ref[:d, :],
                 preferred_element_type=jnp.float32)
    t1 += jnp.dot(onehot, uw, preferred_element_type=jnp.float32)
    t1_ref[...] = t1 + b1_ref[...]


def _tc_post_kernel(t1_ref, psum_ref, pcnt_ref, w1e_ref, w2_ref, b2_ref,
                    out_ref):
    sums = psum_ref[0] + psum_ref[1]
    cnt = pcnt_ref[0, :, 0:1] + pcnt_ref[1, :, 0:1]
    agg = sums / jnp.maximum(cnt, 1.0)
    h = t1_ref[...] + jnp.dot(agg, w1e_ref[...],
                              preferred_element_type=jnp.float32)
    h = jnp.maximum(h, 0.0)
    out = jnp.dot(h, w2_ref[...], preferred_element_type=jnp.float32)
    out_ref[...] = jnp.maximum(out + b2_ref[...], 0.0)


def kernel(x, edge_index, edge_attr, u, batch, W1, b1, W2, b2):
    n, d = x.shape
    e, de = edge_attr.shape
    g = u.shape[0]
    lat = W2.shape[1]

    n_pad = -(-n // (8 * NS)) * (8 * NS)
    zeros_init = jnp.zeros((n_pad // NS, de), dtype=jnp.float32)
    ones_rows = jnp.ones((CHUNK, de), dtype=jnp.float32)

    psum, pcnt = _sc_scatter_partials(jnp.swapaxes(edge_attr, 0, 1),
                                      edge_index[1], zeros_init,
                                      ones_rows, n_pad, de)

    bn = 1000
    grid = n // bn
    pre = pl.pallas_call(
        functools.partial(_tc_pre_kernel, d=d, de=de, g=g),
        grid=(grid,),
        in_specs=[
            pl.BlockSpec((bn, d), lambda i: (i, 0)),
            pl.BlockSpec((bn, 1), lambda i: (i, 0)),
            pl.BlockSpec((g, d), lambda i: (0, 0)),
            pl.BlockSpec(W1.shape, lambda i: (0, 0)),
            pl.BlockSpec((1, lat), lambda i: (0, 0)),
        ],
        out_specs=pl.BlockSpec((bn, lat), lambda i: (i, 0)),
        out_shape=jax.ShapeDtypeStruct((n, lat), jnp.float32),
    )
    t1 = pre(x, batch.reshape(n, 1), u, W1, b1.reshape(1, lat))

    post = pl.pallas_call(
        _tc_post_kernel,
        grid=(grid,),
        in_specs=[
            pl.BlockSpec((bn, lat), lambda i: (i, 0)),
            pl.BlockSpec((NC, bn, de), lambda i: (0, i, 0)),
            pl.BlockSpec((NC, bn, de), lambda i: (0, i, 0)),
            pl.BlockSpec((de, lat), lambda i: (0, 0)),
            pl.BlockSpec(W2.shape, lambda i: (0, 0)),
            pl.BlockSpec((1, lat), lambda i: (0, 0)),
        ],
        out_specs=pl.BlockSpec((bn, lat), lambda i: (i, 0)),
        out_shape=jax.ShapeDtypeStruct((n, lat), jnp.float32),
    )
    return post(t1, psum, pcnt, W1[d:d + de], W2, b2.reshape(1, lat))


# speedup vs baseline: 2.0199x; 1.1213x over previous
import functools

import jax
import jax.numpy as jnp
from jax import lax
from jax.experimental import pallas as pl
from jax.experimental.pallas import tpu as pltpu
from jax.experimental.pallas import tpu_sc as plsc

NC = 2
NS = 16
CHUNK = 1000


def _sc_scatter_partials(ea_t, dest, zeros_init, ones_rows, n_pad, de):
    e = dest.shape[0]
    nw = NC * NS
    epw = e // nw
    n_chunks = epw // CHUNK
    rows_per_tile = n_pad // NS

    mesh = plsc.VectorSubcoreMesh(
        core_axis_name="c", subcore_axis_name="s",
        num_cores=NC, num_subcores=NS)

    @functools.partial(
        pl.kernel,
        out_type=[
            jax.ShapeDtypeStruct((NC, n_pad, de), jnp.float32),
            jax.ShapeDtypeStruct((NC, n_pad, de), jnp.float32),
        ],
        mesh=mesh,
        compiler_params=pltpu.CompilerParams(use_tc_tiling_on_sc=False,
                                             needs_layout_passes=False),
        scratch_types=[
            pltpu.VMEM_SHARED((n_pad, de), jnp.float32),
            pltpu.VMEM_SHARED((n_pad, de), jnp.float32),
            pltpu.VMEM((2, de * CHUNK), jnp.float32),
            pltpu.VMEM((2, CHUNK, de), jnp.float32),
            pltpu.VMEM((n_chunks, CHUNK), jnp.int32),
            pltpu.VMEM((CHUNK, de), jnp.float32),
            pltpu.SemaphoreType.DMA((2,)),
            pltpu.SemaphoreType.DMA((2,)),
        ],
    )
    def body(attr_hbm, idx_hbm, zeros_hbm, ones_hbm, psum, pcnt,
             sums_sh, cnt_sh, col_v, attr_v, idx_v, ones_v, lsem, ssem):
        c = lax.axis_index("c")
        s = lax.axis_index("s")
        w = c * NS + s

        rbase = s * rows_per_tile
        pltpu.sync_copy(zeros_hbm,
                        sums_sh.at[pl.ds(rbase, rows_per_tile)])
        pltpu.sync_copy(zeros_hbm,
                        cnt_sh.at[pl.ds(rbase, rows_per_tile)])
        pltpu.sync_copy(ones_hbm, ones_v)
        plsc.subcore_barrier()

        ebase = w * epw
        idx0 = lax.iota(jnp.int32, 16) * CHUNK
        unroll = 8

        def start_load(k):
            b = k % 2
            base = ebase + k * CHUNK
            hs = [pltpu.async_copy(attr_hbm.at[f, pl.ds(base, CHUNK)],
                                   col_v.at[b, pl.ds(f * CHUNK, CHUNK)],
                                   lsem.at[b])
                  for f in range(de)]
            hs.append(pltpu.async_copy(idx_hbm.at[pl.ds(base, CHUNK)],
                                       idx_v.at[k], lsem.at[b]))
            return hs

        def assemble(b):
            src = col_v.at[b]
            dst = attr_v.at[b]

            def go(i, idxvec):
                rows = [plsc.load_gather(src, [idxvec + uu])
                        for uu in range(unroll)]
                for uu in range(unroll):
                    dst[i * unroll + uu] = rows[uu]
                return idxvec + unroll

            lax.fori_loop(0, CHUNK // unroll, go, idx0)

        def start_scatter(k):
            b = k % 2
            return [pltpu.async_copy(attr_v.at[b], sums_sh.at[idx_v.at[k]],
                                     ssem.at[b], add=True),
                    pltpu.async_copy(ones_v, cnt_sh.at[idx_v.at[k]],
                                     ssem.at[b], add=True)]

        loads = {0: start_load(0)}
        scats = {}
        for k in range(n_chunks):
            b = k % 2
            for h in loads.pop(k):
                h.wait()
            if k + 1 < n_chunks:
                loads[k + 1] = start_load(k + 1)
            if k - 2 in scats:
                for h in scats.pop(k - 2):
                    h.wait()
            assemble(b)
            scats[k] = start_scatter(k)
        for hs in scats.values():
            for h in hs:
                h.wait()

        plsc.subcore_barrier()

        pltpu.sync_copy(sums_sh.at[pl.ds(rbase, rows_per_tile)],
                        psum.at[c, pl.ds(rbase, rows_per_tile)])
        pltpu.sync_copy(cnt_sh.at[pl.ds(rbase, rows_per_tile)],
                        pcnt.at[c, pl.ds(rbase, rows_per_tile)])

    return body(ea_t, dest, zeros_init, ones_rows)


def _tc_dest_kernel(ei_ref, dest_ref):
    dest_ref[...] = ei_ref[1, :]


def _extract_dest(edge_index):
    e = edge_index.shape[1]
    return pl.pallas_call(
        _tc_dest_kernel,
        out_shape=jax.ShapeDtypeStruct((e,), jnp.int32),
    )(edge_index)


def _tc_pre_kernel(x_ref, batch_ref, u_ref, w1_ref, b1_ref, t1_ref,
                   *, d, de, g):
    onehot = (batch_ref[...] ==
              lax.broadcasted_iota(jnp.int32, (1, g), 1)).astype(jnp.float32)
    uw = jnp.dot(u_ref[...], w1_ref[d + de:, :],
                 preferred_element_type=jnp.float32)
    t1 = jnp.dot(x_ref[...], w1_ref[:d, :],
                 preferred_element_type=jnp.float32)
    t1 += jnp.dot(onehot, uw, preferred_element_type=jnp.float32)
    t1_ref[...] = t1 + b1_ref[...]


def _tc_post_kernel(t1_ref, psum_ref, pcnt_ref, w1e_ref, w2_ref, b2_ref,
                    out_ref):
    sums = psum_ref[0] + psum_ref[1]
    cnt = pcnt_ref[0, :, 0:1] + pcnt_ref[1, :, 0:1]
    agg = sums / jnp.maximum(cnt, 1.0)
    h = t1_ref[...] + jnp.dot(agg, w1e_ref[...],
                              preferred_element_type=jnp.float32)
    h = jnp.maximum(h, 0.0)
    out = jnp.dot(h, w2_ref[...], preferred_element_type=jnp.float32)
    out_ref[...] = jnp.maximum(out + b2_ref[...], 0.0)


def kernel(x, edge_index, edge_attr, u, batch, W1, b1, W2, b2):
    n, d = x.shape
    e, de = edge_attr.shape
    g = u.shape[0]
    lat = W2.shape[1]

    n_pad = -(-n // (8 * NS)) * (8 * NS)
    zeros_init = jnp.zeros((n_pad // NS, de), dtype=jnp.float32)
    ones_rows = jnp.ones((CHUNK, de), dtype=jnp.float32)

    psum, pcnt = _sc_scatter_partials(jnp.swapaxes(edge_attr, 0, 1),
                                      _extract_dest(edge_index), zeros_init,
                                      ones_rows, n_pad, de)

    bn = 1000
    grid = n // bn
    pre = pl.pallas_call(
        functools.partial(_tc_pre_kernel, d=d, de=de, g=g),
        grid=(grid,),
        in_specs=[
            pl.BlockSpec((bn, d), lambda i: (i, 0)),
            pl.BlockSpec((bn, 1), lambda i: (i, 0)),
            pl.BlockSpec((g, d), lambda i: (0, 0)),
            pl.BlockSpec(W1.shape, lambda i: (0, 0)),
            pl.BlockSpec((1, lat), lambda i: (0, 0)),
        ],
        out_specs=pl.BlockSpec((bn, lat), lambda i: (i, 0)),
        out_shape=jax.ShapeDtypeStruct((n, lat), jnp.float32),
    )
    t1 = pre(x, batch.reshape(n, 1), u, W1, b1.reshape(1, lat))

    post = pl.pallas_call(
        _tc_post_kernel,
        grid=(grid,),
        in_specs=[
            pl.BlockSpec((bn, lat), lambda i: (i, 0)),
            pl.BlockSpec((NC, bn, de), lambda i: (0, i, 0)),
            pl.BlockSpec((NC, bn, de), lambda i: (0, i, 0)),
            pl.BlockSpec((de, lat), lambda i: (0, 0)),
            pl.BlockSpec(W2.shape, lambda i: (0, 0)),
            pl.BlockSpec((1, lat), lambda i: (0, 0)),
        ],
        out_specs=pl.BlockSpec((bn, lat), lambda i: (i, 0)),
        out_shape=jax.ShapeDtypeStruct((n, lat), jnp.float32),
    )
    return post(t1, psum, pcnt, W1[d:d + de], W2, b2.reshape(1, lat))
